# baseline (speedup 1.0000x reference)
import functools

import jax
import jax.numpy as jnp
from jax import lax
from jax.experimental import pallas as pl
from jax.experimental.pallas import tpu as pltpu, tpu_sc as plsc

_NUM_CLASSES = 100000
_DIM = 64
_BATCH = 16384

_INFO = plsc.get_sparse_core_info()
_NC = _INFO.num_cores
_NS = _INFO.num_subcores
_NW = _NC * _NS
_BPW = _BATCH // _NW
_CHUNK = 128
_NCHUNK = _BPW // _CHUNK
_LANES = 16
_DCH = _DIM // _LANES


def _body(centers_hbm, y_hbm, f_hbm, out_hbm, idx_v, rows_v, f_v, acc_v,
          gsem, fsem):
    c = lax.axis_index("c")
    s = lax.axis_index("s")
    wid = s * _NC + c
    base = wid * _BPW

    pltpu.sync_copy(y_hbm.at[wid], idx_v)
    fcopy = pltpu.make_async_copy(f_hbm.at[pl.ds(base, _BPW)], f_v, fsem)
    fcopy.start()

    gathers = []
    for j in range(_NCHUNK):
        g = pltpu.make_async_copy(
            centers_hbm.at[idx_v.at[j]],
            rows_v.at[pl.ds(j * _CHUNK, _CHUNK)],
            gsem,
        )
        g.start()
        gathers.append(g)
    fcopy.wait()
    for g in gathers:
        g.wait()

    zero = jnp.zeros((_LANES,), jnp.float32)

    def step(r, accs):
        new = []
        for k in range(_DCH):
            d = f_v[r, pl.ds(k * _LANES, _LANES)] - rows_v[r, pl.ds(k * _LANES, _LANES)]
            new.append(accs[k] + d * d)
        return tuple(new)

    accs = lax.fori_loop(0, _BPW, step, (zero,) * _DCH, unroll=4)
    total = accs[0] + accs[1] + accs[2] + accs[3]

    acc_v[...] = total
    pltpu.sync_copy(acc_v, out_hbm.at[wid])


_sc_call = pl.kernel(
    _body,
    out_type=jax.ShapeDtypeStruct((_NW, _LANES), jnp.float32),
    mesh=plsc.VectorSubcoreMesh(core_axis_name="c", subcore_axis_name="s"),
    compiler_params=pltpu.CompilerParams(use_tc_tiling_on_sc=False),
    scratch_types=[
        pltpu.VMEM((_NCHUNK, _CHUNK), jnp.int32),
        pltpu.VMEM((_BPW, _DIM), jnp.float32),
        pltpu.VMEM((_BPW, _DIM), jnp.float32),
        pltpu.VMEM((_LANES,), jnp.float32),
        pltpu.SemaphoreType.DMA,
        pltpu.SemaphoreType.DMA,
    ],
)


@jax.jit
def kernel(f, y, centers):
    yr = y.astype(jnp.int32).reshape(_NW, _NCHUNK, _CHUNK)
    partials = _sc_call(centers, yr, f)
    return jnp.sum(partials) * (1.0 / _BATCH)


# speedup vs baseline: 1.8659x; 1.8659x over previous
import jax
import jax.numpy as jnp
from jax import lax
from jax.experimental import pallas as pl
from jax.experimental.pallas import tpu as pltpu, tpu_sc as plsc

_NUM_CLASSES = 100000
_DIM = 64
_BATCH = 16384

_INFO = plsc.get_sparse_core_info()
_NC = _INFO.num_cores
_NS = _INFO.num_subcores
_NW = _NC * _NS
_CPT = _DIM // _NW
_LANES = 16
_ICH = 4096
_NICH = _BATCH // _ICH
_UNROLL = 4


def _body(ct_hbm, y_hbm, ft_hbm, out_hbm, col_v, idx_v, f_v, acc_v):
    c = lax.axis_index("c")
    s = lax.axis_index("s")
    wid = s * _NC + c

    accs = (jnp.zeros((_LANES,), jnp.float32),) * _UNROLL
    for ci in range(_CPT):
        col = wid * _CPT + ci
        pltpu.sync_copy(ct_hbm.at[col], col_v)
        for ch in range(_NICH):
            base = ch * _ICH
            pltpu.sync_copy(y_hbm.at[pl.ds(base, _ICH)], idx_v)
            pltpu.sync_copy(ft_hbm.at[col, pl.ds(base, _ICH)], f_v)

            def step(k, accs):
                new = []
                for u in range(_UNROLL):
                    off = (k * _UNROLL + u) * _LANES
                    g = plsc.load_gather(col_v, [idx_v[pl.ds(off, _LANES)]])
                    d = f_v[pl.ds(off, _LANES)] - g
                    new.append(accs[u] + d * d)
                return tuple(new)

            accs = lax.fori_loop(0, _ICH // (_UNROLL * _LANES), step, accs)

    total = (accs[0] + accs[1]) + (accs[2] + accs[3])
    acc_v[...] = total
    pltpu.sync_copy(acc_v, out_hbm.at[wid])


_sc_call = pl.kernel(
    _body,
    out_type=jax.ShapeDtypeStruct((_NW, _LANES), jnp.float32),
    mesh=plsc.VectorSubcoreMesh(core_axis_name="c", subcore_axis_name="s"),
    compiler_params=pltpu.CompilerParams(needs_layout_passes=False),
    scratch_types=[
        pltpu.VMEM((_NUM_CLASSES,), jnp.float32),
        pltpu.VMEM((_ICH,), jnp.int32),
        pltpu.VMEM((_ICH,), jnp.float32),
        pltpu.VMEM((_LANES,), jnp.float32),
    ],
)


@jax.jit
def kernel(f, y, centers):
    partials = _sc_call(centers.T, y.astype(jnp.int32), f.T)
    return jnp.sum(partials) * (1.0 / _BATCH)


# speedup vs baseline: 2.3008x; 1.2331x over previous
import jax
import jax.numpy as jnp
from jax import lax
from jax.experimental import pallas as pl
from jax.experimental.pallas import tpu as pltpu, tpu_sc as plsc

_NUM_CLASSES = 100000
_DIM = 64
_BATCH = 16384

_INFO = plsc.get_sparse_core_info()
_NC = _INFO.num_cores
_NS = _INFO.num_subcores
_NW = _NC * _NS
_CPT = _DIM // _NW
_LANES = 16
_FCH = 4096
_NFCH = _BATCH // _FCH
_UNROLL = 8


def _body(ct_hbm, y_hbm, ft_hbm, out_hbm, col_v, idx_v, f_v, acc_v,
          csem, isem, fsem):
    c = lax.axis_index("c")
    s = lax.axis_index("s")
    wid = s * _NC + c

    idx_cp = pltpu.make_async_copy(y_hbm, idx_v, isem)
    idx_cp.start()

    col_cps = []
    for ci in range(_CPT):
        col_cps.append(pltpu.make_async_copy(
            ct_hbm.at[wid * _CPT + ci], col_v, csem))
    col_cps[0].start()

    def f_copy(col, ch, buf):
        return pltpu.make_async_copy(
            ft_hbm.at[col, pl.ds(ch * _FCH, _FCH)], f_v.at[buf], fsem.at[buf])

    f_copy(wid * _CPT, 0, 0).start()
    idx_cp.wait()

    accs = (jnp.zeros((_LANES,), jnp.float32),) * _UNROLL
    for ci in range(_CPT):
        col = wid * _CPT + ci
        col_cps[ci].wait()
        for ch in range(_NFCH):
            buf = ch % 2
            if ch + 1 < _NFCH:
                f_copy(col, ch + 1, 1 - buf).start()
            elif ci + 1 < _CPT:
                f_copy(col + 1, 0, 1 - buf).start()
            f_copy(col, ch, buf).wait()
            base = ch * _FCH

            def step(k, accs):
                new = []
                for u in range(_UNROLL):
                    off = base + (k * _UNROLL + u) * _LANES
                    g = plsc.load_gather(col_v, [idx_v[pl.ds(off, _LANES)]])
                    d = f_v[buf, pl.ds(off - base, _LANES)] - g
                    new.append(accs[u] + d * d)
                return tuple(new)

            accs = lax.fori_loop(0, _FCH // (_UNROLL * _LANES), step, accs)
        if ci + 1 < _CPT:
            col_cps[ci + 1].start()

    total = accs[0]
    for u in range(1, _UNROLL):
        total = total + accs[u]
    acc_v[...] = total
    pltpu.sync_copy(acc_v, out_hbm.at[wid])


_sc_call = pl.kernel(
    _body,
    out_type=jax.ShapeDtypeStruct((_NW, _LANES), jnp.float32),
    mesh=plsc.VectorSubcoreMesh(core_axis_name="c", subcore_axis_name="s"),
    compiler_params=pltpu.CompilerParams(needs_layout_passes=False),
    scratch_types=[
        pltpu.VMEM((_NUM_CLASSES,), jnp.float32),
        pltpu.VMEM((_BATCH,), jnp.int32),
        pltpu.VMEM((2, _FCH), jnp.float32),
        pltpu.VMEM((_LANES,), jnp.float32),
        pltpu.SemaphoreType.DMA,
        pltpu.SemaphoreType.DMA,
        pltpu.SemaphoreType.DMA((2,)),
    ],
)


@jax.jit
def kernel(f, y, centers):
    partials = _sc_call(centers.T, y.astype(jnp.int32), f.T)
    return jnp.sum(partials) * (1.0 / _BATCH)


# speedup vs baseline: 2.3110x; 1.0044x over previous
import jax
import jax.numpy as jnp
from jax import lax
from jax.experimental import pallas as pl
from jax.experimental.pallas import tpu as pltpu, tpu_sc as plsc

_NUM_CLASSES = 100000
_DIM = 64
_BATCH = 16384

_INFO = plsc.get_sparse_core_info()
_NC = _INFO.num_cores
_NS = _INFO.num_subcores
_NW = _NC * _NS
_CPT = _DIM // _NW
_LANES = 16
_FCH = 4096
_NFCH = _BATCH // _FCH
_UNROLL = 8


def _body(ct_hbm, y_hbm, ft_hbm, out_hbm, col_v, idx_v, f_v, acc_v,
          csem, isem, fsem):
    c = lax.axis_index("c")
    s = lax.axis_index("s")
    wid = s * _NC + c

    idx_cp = pltpu.make_async_copy(y_hbm, idx_v, isem)
    idx_cp.start()

    col_cps = []
    for ci in range(_CPT):
        col_cps.append(pltpu.make_async_copy(
            ct_hbm.at[wid * _CPT + ci], col_v, csem))
    col_cps[0].start()

    def f_copy(col, ch, buf):
        return pltpu.make_async_copy(
            ft_hbm.at[col, pl.ds(ch * _FCH, _FCH)], f_v.at[buf], fsem.at[buf])

    f_copy(wid * _CPT, 0, 0).start()
    idx_cp.wait()

    accs = (jnp.zeros((_LANES,), jnp.float32),) * _UNROLL
    for ci in range(_CPT):
        col = wid * _CPT + ci
        col_cps[ci].wait()
        for ch in range(_NFCH):
            buf = ch % 2
            if ch + 1 < _NFCH:
                f_copy(col, ch + 1, 1 - buf).start()
            elif ci + 1 < _CPT:
                f_copy(col + 1, 0, 1 - buf).start()
            f_copy(col, ch, buf).wait()
            base = ch * _FCH

            def step(k, accs):
                new = []
                for u in range(_UNROLL):
                    off = base + (k * _UNROLL + u) * _LANES
                    g = plsc.load_gather(col_v, [idx_v[pl.ds(off, _LANES)]])
                    d = f_v[buf, pl.ds(off - base, _LANES)] - g
                    new.append(accs[u] + d * d)
                return tuple(new)

            accs = lax.fori_loop(0, _FCH // (_UNROLL * _LANES), step, accs)
        if ci + 1 < _CPT:
            col_cps[ci + 1].start()

    total = accs[0]
    for u in range(1, _UNROLL):
        total = total + accs[u]
    acc_v[...] = total
    pltpu.sync_copy(acc_v, out_hbm.at[wid])


_sc_call = pl.kernel(
    _body,
    out_type=jax.ShapeDtypeStruct((_NW, _LANES), jnp.float32),
    mesh=plsc.VectorSubcoreMesh(core_axis_name="c", subcore_axis_name="s"),
    compiler_params=pltpu.CompilerParams(
        needs_layout_passes=False, skip_device_barrier=True),
    scratch_types=[
        pltpu.VMEM((_NUM_CLASSES,), jnp.float32),
        pltpu.VMEM((_BATCH,), jnp.int32),
        pltpu.VMEM((2, _FCH), jnp.float32),
        pltpu.VMEM((_LANES,), jnp.float32),
        pltpu.SemaphoreType.DMA,
        pltpu.SemaphoreType.DMA,
        pltpu.SemaphoreType.DMA((2,)),
    ],
)


@jax.jit
def kernel(f, y, centers):
    partials = _sc_call(centers.T, y.astype(jnp.int32), f.T)
    return jnp.sum(partials) * (1.0 / _BATCH)


# speedup vs baseline: 2.3249x; 1.0060x over previous
import jax
import jax.numpy as jnp
from jax import lax
from jax.experimental import pallas as pl
from jax.experimental.pallas import tpu as pltpu, tpu_sc as plsc

_NUM_CLASSES = 100000
_DIM = 64
_BATCH = 16384

_INFO = plsc.get_sparse_core_info()
_NC = _INFO.num_cores
_NS = _INFO.num_subcores
_NW = _NC * _NS
_CPT = _DIM // _NW
_LANES = 16
_FCH = 4096
_NFCH = _BATCH // _FCH
_UNROLL = 8


def _body(ct_hbm, y_hbm, ft_hbm, out_hbm, col_v, idx_v, f_v, acc_v,
          csem, isem, fsem):
    c = lax.axis_index("c")
    s = lax.axis_index("s")
    wid = s * _NC + c

    idx_cp = pltpu.make_async_copy(y_hbm, idx_v, isem)
    idx_cp.start()

    col_cps = []
    for ci in range(_CPT):
        col_cps.append(pltpu.make_async_copy(
            ct_hbm.at[wid * _CPT + ci], col_v, csem))
    col_cps[0].start()

    def f_copy(col, ch, buf):
        return pltpu.make_async_copy(
            ft_hbm.at[col, pl.ds(ch * _FCH, _FCH)], f_v.at[buf], fsem.at[buf])

    f_copy(wid * _CPT, 0, 0).start()
    idx_cp.wait()

    accs = (jnp.zeros((_LANES,), jnp.float32),) * _UNROLL
    for ci in range(_CPT):
        col = wid * _CPT + ci
        col_cps[ci].wait()
        for ch in range(_NFCH):
            buf = ch % 2
            if ch + 1 < _NFCH:
                f_copy(col, ch + 1, 1 - buf).start()
            elif ci + 1 < _CPT:
                f_copy(col + 1, 0, 1 - buf).start()
            f_copy(col, ch, buf).wait()
            base = ch * _FCH

            def step(off, accs, base=base, buf=buf):
                new = []
                for u in range(_UNROLL):
                    o = off + u * _LANES
                    g = plsc.load_gather(
                        col_v, [idx_v[pl.ds(base + o, _LANES)]])
                    d = f_v[buf, pl.ds(o, _LANES)] - g
                    new.append(accs[u] + d * d)
                return tuple(new)

            accs = plsc.parallel_loop(
                0, _FCH, step=_UNROLL * _LANES, carry=accs)(step)

        if ci + 1 < _CPT:
            col_cps[ci + 1].start()

    total = accs[0]
    for u in range(1, _UNROLL):
        total = total + accs[u]
    acc_v[...] = total
    pltpu.sync_copy(acc_v, out_hbm.at[wid])


_sc_call = pl.kernel(
    _body,
    out_type=jax.ShapeDtypeStruct((_NW, _LANES), jnp.float32),
    mesh=plsc.VectorSubcoreMesh(core_axis_name="c", subcore_axis_name="s"),
    compiler_params=pltpu.CompilerParams(
        needs_layout_passes=False, skip_device_barrier=True),
    scratch_types=[
        pltpu.VMEM((_NUM_CLASSES,), jnp.float32),
        pltpu.VMEM((_BATCH,), jnp.int32),
        pltpu.VMEM((2, _FCH), jnp.float32),
        pltpu.VMEM((_LANES,), jnp.float32),
        pltpu.SemaphoreType.DMA,
        pltpu.SemaphoreType.DMA,
        pltpu.SemaphoreType.DMA((2,)),
    ],
)


@jax.jit
def kernel(f, y, centers):
    partials = _sc_call(centers.T, y.astype(jnp.int32), f.T)
    return jnp.sum(partials) * (1.0 / _BATCH)


# speedup vs baseline: 2.3878x; 1.0271x over previous
import jax
import jax.numpy as jnp
from jax import lax
from jax.experimental import pallas as pl
from jax.experimental.pallas import tpu as pltpu, tpu_sc as plsc

_NUM_CLASSES = 100000
_DIM = 64
_BATCH = 16384

_INFO = plsc.get_sparse_core_info()
_NC = _INFO.num_cores
_NS = _INFO.num_subcores
_NW = _NC * _NS
_CPT = _DIM // _NW
_LANES = 16
_FCH = 4096
_NCH = _BATCH // _FCH * _CPT
_UNROLL = 8


def _body(ct_hbm, y_hbm, ft_hbm, out_hbm, col_v, idx_v, f_v, acc_v,
          csem, isem, fsem):
    c = lax.axis_index("c")
    s = lax.axis_index("s")
    wid = s * _NC + c
    col0 = wid * _CPT
    half = _NCH // _CPT

    def col_copy(ci):
        return pltpu.make_async_copy(ct_hbm.at[col0 + ci], col_v, csem)

    def f_copy(t, buf):
        col = col0 + lax.div(t, half)
        ch = lax.rem(t, half)
        return pltpu.make_async_copy(
            ft_hbm.at[col, pl.ds(ch * _FCH, _FCH)],
            f_v.at[pl.ds(buf * _FCH, _FCH)], fsem.at[buf])

    pltpu.make_async_copy(y_hbm, idx_v, isem).start()
    col_copy(0).start()
    f_copy(0, 0).start()
    pltpu.make_async_copy(y_hbm, idx_v, isem).wait()

    accs = (jnp.zeros((_LANES,), jnp.float32),) * _UNROLL

    def chunk(t, accs):
        buf = lax.rem(t, 2)

        @pl.when(t == 0)
        def _():
            col_copy(0).wait()

        @pl.when(t == half)
        def _():
            col_copy(1).start()
            col_copy(1).wait()

        @pl.when(t + 1 < _NCH)
        def _():
            f_copy(t + 1, 1 - buf).start()

        f_copy(t, buf).wait()
        ibase = lax.rem(t, half) * _FCH
        fbase = buf * _FCH

        def step(off, accs):
            new = []
            for u in range(_UNROLL):
                o = off + u * _LANES
                g = plsc.load_gather(col_v, [idx_v[pl.ds(ibase + o, _LANES)]])
                d = f_v[pl.ds(fbase + o, _LANES)] - g
                new.append(accs[u] + d * d)
            return tuple(new)

        return plsc.parallel_loop(
            0, _FCH, step=_UNROLL * _LANES, carry=accs)(step)

    accs = lax.fori_loop(0, _NCH, chunk, accs)

    total = accs[0]
    for u in range(1, _UNROLL):
        total = total + accs[u]
    acc_v[...] = total
    pltpu.sync_copy(acc_v, out_hbm.at[wid])


_sc_call = pl.kernel(
    _body,
    out_type=jax.ShapeDtypeStruct((_NW, _LANES), jnp.float32),
    mesh=plsc.VectorSubcoreMesh(core_axis_name="c", subcore_axis_name="s"),
    compiler_params=pltpu.CompilerParams(
        needs_layout_passes=False, skip_device_barrier=True),
    scratch_types=[
        pltpu.VMEM((_NUM_CLASSES,), jnp.float32),
        pltpu.VMEM((_BATCH,), jnp.int32),
        pltpu.VMEM((2 * _FCH,), jnp.float32),
        pltpu.VMEM((_LANES,), jnp.float32),
        pltpu.SemaphoreType.DMA,
        pltpu.SemaphoreType.DMA,
        pltpu.SemaphoreType.DMA((2,)),
    ],
)


@jax.jit
def kernel(f, y, centers):
    partials = _sc_call(centers.T, y.astype(jnp.int32), f.T)
    return jnp.sum(partials) * (1.0 / _BATCH)
